# baseline (speedup 1.0000x reference)
import jax
import jax.numpy as jnp
from jax import lax
from jax.experimental import pallas as pl
from jax.experimental.pallas import tpu as pltpu
from jax.experimental.pallas import tpu_sc as plsc

_D = 512
_ROWS_PER_W = 8


def _body(l1_hbm, l0_hbm, out_hbm, buf):
    wid = lax.axis_index("s") * 2 + lax.axis_index("c")

    @pl.when(wid < 16)
    def _():
        base = wid * _ROWS_PER_W
        pltpu.sync_copy(l1_hbm.at[pl.ds(base, _ROWS_PER_W)], buf)
        pltpu.sync_copy(buf, out_hbm.at[pl.ds(base, _ROWS_PER_W)])

    @pl.when(wid >= 16)
    def _():
        m = wid - 16
        for t in range(4):
            pltpu.sync_copy(
                l0_hbm.at[pl.ds(16 * m + 4 * t, 2)],
                buf.at[pl.ds(2 * t, 2)],
            )
        pltpu.sync_copy(buf, out_hbm.at[pl.ds(128 + m * _ROWS_PER_W, _ROWS_PER_W)])


@jax.jit
def kernel(layer1, layer0):
    mesh = plsc.VectorSubcoreMesh(core_axis_name="c", subcore_axis_name="s")
    f = pl.kernel(
        _body,
        out_type=jax.ShapeDtypeStruct((256, _D), jnp.float32),
        mesh=mesh,
        scratch_types=[pltpu.VMEM((_ROWS_PER_W, _D), jnp.float32)],
    )
    return f(layer1, layer0)
